# baseline (speedup 1.0000x reference)
import jax
import jax.numpy as jnp
from jax import lax
from jax.experimental import pallas as pl
from jax.experimental.pallas import tpu as pltpu
from jax.experimental.pallas import tpu_sc as plsc

N_NODES = 10000
N_EDGES = 320000
H = 128

NC = 2
NS = 16
L = 16
NW = NC * NS
HW = H // 2
ECH = 64
NCH_ALL = N_EDGES // ECH
NCH_BASE = NCH_ALL // NW
NCH_REM = NCH_ALL % NW
NTRIP = (NCH_BASE + 1 + 2) // 3
RPT = 624
TAIL0 = NS * RPT
TAILN = N_NODES - TAIL0


def _tables_body(x_ref, a_ref, b_ref, xa_ref, xb_ref, xh_ref):
    x = x_ref[...]
    xa_ref[...] = jnp.dot(x, a_ref[...], preferred_element_type=jnp.float32)
    xb_ref[...] = jnp.dot(x, b_ref[...], preferred_element_type=jnp.float32)
    xh_ref[...] = x * 0.5


def _tables(x, a, b):
    n = x.shape[0]
    return pl.pallas_call(
        _tables_body,
        out_shape=(
            jax.ShapeDtypeStruct((n, H), jnp.float32),
            jax.ShapeDtypeStruct((n, H), jnp.float32),
            jax.ShapeDtypeStruct((n, H), jnp.float32),
        ),
    )(x, a, b)


def _efc_body(ef_ref, c_ref, b_ref, o_ref):
    y = (
        jnp.dot(ef_ref[...], c_ref[...], preferred_element_type=jnp.float32)
        + b_ref[...]
    ).astype(jnp.bfloat16)
    lo = lax.bitcast_convert_type(y[:, :HW], jnp.uint16).astype(jnp.uint32)
    hi = lax.bitcast_convert_type(y[:, HW:], jnp.uint16).astype(jnp.uint32)
    o_ref[...] = lax.bitcast_convert_type(
        lo | (hi << jnp.uint32(16)), jnp.int32)


def _efc(ef, c, b1):
    e = ef.shape[0]
    be = 4000
    grid = e // be
    return pl.pallas_call(
        _efc_body,
        grid=(grid,),
        in_specs=[
            pl.BlockSpec((be, H), lambda i: (i, 0)),
            pl.BlockSpec((H, H), lambda i: (0, 0)),
            pl.BlockSpec((1, H), lambda i: (0, 0)),
        ],
        out_specs=pl.BlockSpec((be, HW), lambda i: (i, 0)),
        out_shape=jax.ShapeDtypeStruct((e, HW), jnp.int32),
    )(ef, c, b1.reshape(1, H))


def _sc_body(xa, xb, efc, s_hbm, r_hbm, xh_hbm, out_hbm,
             sidx0, sidx1, sidx2, ridx0, ridx1, ridx2,
             ebuf0, ebuf1, ebuf2, rsc0, rsc1, rsc2, acc0, acc1, acc2, aggr,
             semi0, semi1, semi2, sema0, sema1, sema2,
             semb0, semb1, semb2, semc0, semc1, semc2, ssem):
    c = lax.axis_index("c")
    s = lax.axis_index("s")
    wid = c * NS + s
    sidxs = (sidx0, sidx1, sidx2)
    ridxs = (ridx0, ridx1, ridx2)
    ebufs = (ebuf0, ebuf1, ebuf2)
    rscs = (rsc0, rsc1, rsc2)
    accs = (acc0, acc1, acc2)
    semis = (semi0, semi1, semi2)
    semas = (sema0, sema1, sema2)
    sembs = (semb0, semb1, semb2)
    semcs = (semc0, semc1, semc2)

    row0 = s * RPT
    pltpu.async_copy(xh_hbm.at[pl.ds(row0, RPT)], aggr.at[pl.ds(row0, RPT)],
                     ssem).wait()

    @pl.when(s == 0)
    def _seed_tail():
        pltpu.async_copy(xh_hbm.at[pl.ds(TAIL0, TAILN)],
                         aggr.at[pl.ds(TAIL0, TAILN)], ssem).wait()

    plsc.subcore_barrier()

    nchunk_w = NCH_BASE + jnp.where(wid < NCH_REM, 1, 0)

    def ebase(k):
        return (k * NW + wid) * ECH

    def issue_idx(k, i):
        pltpu.async_copy(s_hbm.at[pl.ds(ebase(k), ECH)], sidxs[i], semis[i])
        pltpu.async_copy(r_hbm.at[pl.ds(ebase(k), ECH)], ridxs[i], semis[i])

    def wait_idx(k, i):
        pltpu.make_async_copy(s_hbm.at[pl.ds(ebase(k), ECH)], sidxs[i],
                              semis[i]).wait()
        pltpu.make_async_copy(r_hbm.at[pl.ds(ebase(k), ECH)], ridxs[i],
                              semis[i]).wait()

    def issue_xa(k, i):
        pltpu.async_copy(xa.at[sidxs[i]], accs[i], semas[i])
        pltpu.async_copy(efc.at[pl.ds(ebase(k), ECH)], ebufs[i], semas[i])

    def wait_xa(k, i):
        pltpu.make_async_copy(xa.at[sidxs[i]], accs[i], semas[i]).wait()
        pltpu.make_async_copy(efc.at[pl.ds(ebase(k), ECH)], ebufs[i],
                              semas[i]).wait()

    def issue_adds(k, i):
        pltpu.async_copy(xb.at[ridxs[i]], accs[i], sembs[i], add=True)

    def wait_adds(k, i):
        pltpu.make_async_copy(xb.at[ridxs[i]], accs[i], sembs[i]).wait()

    def wait_scatter(i):
        pltpu.make_async_copy(accs[i], aggr.at[rscs[i]], semcs[i]).wait()

    issue_idx(0, 0)
    issue_idx(1, 1)
    issue_idx(2, 2)
    wait_idx(0, 0)
    issue_xa(0, 0)
    wait_xa(0, 0)
    issue_adds(0, 0)
    wait_idx(1, 1)
    issue_xa(1, 1)

    def sub_body(k, i):
        i1 = (i + 1) % 3
        i2 = (i + 2) % 3

        @pl.when(k + 2 < nchunk_w)
        def _start_xa():
            wait_idx(k + 2, i2)

            @pl.when(k >= 1)
            def _prev_scatter_done():
                wait_scatter(i2)

            issue_xa(k + 2, i2)

        @pl.when(k + 1 < nchunk_w)
        def _start_adds():
            wait_xa(k + 1, i1)
            issue_adds(k + 1, i1)

        wait_adds(k, i)
        acc_i = accs[i]
        ebuf_i = ebufs[i]
        rsc_i = rscs[i]

        for j in range(ECH // L):
            rsc_i[pl.ds(j * L, L)] = ridxs[i][pl.ds(j * L, L)]

        mhi = jnp.int32(-65536)

        def row(rr, carry2):
            for j in range(HW // L):
                we = ebuf_i[rr, pl.ds(j * L, L)]
                lo = lax.bitcast_convert_type(lax.shift_left(we, 16),
                                              jnp.float32)
                hi = lax.bitcast_convert_type(lax.bitwise_and(we, mhi),
                                              jnp.float32)
                sl_lo = pl.ds(j * L, L)
                sl_hi = pl.ds(HW + j * L, L)
                acc_i[rr, sl_lo] = jnp.maximum(acc_i[rr, sl_lo] + lo, 0.0)
                acc_i[rr, sl_hi] = jnp.maximum(acc_i[rr, sl_hi] + hi, 0.0)
            return carry2

        lax.fori_loop(0, ECH, row, 0, unroll=2)
        pltpu.async_copy(acc_i, aggr.at[rsc_i], semcs[i], add=True)

        @pl.when(k + 3 < nchunk_w)
        def _refill_idx():
            issue_idx(k + 3, i)

    def triple(t, carry):
        k = t * 3

        @pl.when(k < nchunk_w)
        def _first():
            sub_body(k, 0)

        @pl.when(k + 1 < nchunk_w)
        def _mid():
            sub_body(k + 1, 1)

        @pl.when(k + 2 < nchunk_w)
        def _last():
            sub_body(k + 2, 2)

        return carry

    lax.fori_loop(0, NTRIP, triple, 0)

    wait_scatter(0)
    wait_scatter(1)
    wait_scatter(2)

    plsc.subcore_barrier()
    pltpu.async_copy(aggr.at[pl.ds(row0, RPT)],
                     out_hbm.at[c, pl.ds(row0, RPT)], ssem).wait()

    @pl.when(s == 0)
    def _write_tail():
        pltpu.async_copy(aggr.at[pl.ds(TAIL0, TAILN)],
                         out_hbm.at[c, pl.ds(TAIL0, TAILN)], ssem).wait()


def _sc_stage(xa, xb, efc, senders, receivers, xh):
    mesh = plsc.VectorSubcoreMesh(core_axis_name="c", subcore_axis_name="s")
    kfun = pl.kernel(
        _sc_body,
        out_type=jax.ShapeDtypeStruct((NC, N_NODES, H), jnp.float32),
        mesh=mesh,
        scratch_types=(
            [pltpu.VMEM((ECH,), jnp.int32)] * 6
            + [pltpu.VMEM((ECH, HW), jnp.int32)] * 3
            + [pltpu.VMEM((ECH,), jnp.int32)] * 3
            + [pltpu.VMEM((ECH, H), jnp.float32)] * 3
            + [pltpu.VMEM_SHARED((N_NODES, H), jnp.float32)]
            + [pltpu.SemaphoreType.DMA] * 13
        ),
    )
    return kfun(xa, xb, efc, senders, receivers, xh)


def _combine_body(p_ref, o_ref):
    o_ref[...] = p_ref[0] + p_ref[1]


def _combine(partials):
    bn = 2000
    grid = N_NODES // bn
    return pl.pallas_call(
        _combine_body,
        grid=(grid,),
        in_specs=[pl.BlockSpec((NC, bn, H), lambda i: (0, i, 0))],
        out_specs=pl.BlockSpec((bn, H), lambda i: (i, 0)),
        out_shape=jax.ShapeDtypeStruct((N_NODES, H), jnp.float32),
    )(partials)


def kernel(x, senders, receivers, edge_feat, W1, b1):
    senders = senders.astype(jnp.int32)
    receivers = receivers.astype(jnp.int32)
    a = W1[:H]
    b = W1[H:2 * H]
    c = W1[2 * H:]
    xa, xb, xh = _tables(x, a, b)
    efc = _efc(edge_feat, c, b1)
    partials = _sc_stage(xa, xb, efc, senders, receivers, xh)
    return _combine(partials)


# speedup vs baseline: 1.2516x; 1.2516x over previous
import jax
import jax.numpy as jnp
from jax import lax
from jax.experimental import pallas as pl
from jax.experimental.pallas import tpu as pltpu
from jax.experimental.pallas import tpu_sc as plsc

N_NODES = 10000
N_EDGES = 320000
H = 128

NC = 2
NS = 16
L = 16
NW = NC * NS
ECH = 128
NCH_ALL = N_EDGES // ECH
NCH_BASE = NCH_ALL // NW
NCH_REM = NCH_ALL % NW
NTRIP = (NCH_BASE + 1 + 2) // 3
RPT = 624
TAIL0 = NS * RPT
TAILN = N_NODES - TAIL0


def _tables_body(x_ref, a_ref, b_ref, xa_ref, xb_ref, xh_ref):
    x = x_ref[...]
    xa_ref[...] = jnp.dot(x, a_ref[...], preferred_element_type=jnp.float32)
    xb_ref[...] = jnp.dot(x, b_ref[...], preferred_element_type=jnp.float32)
    xh_ref[...] = x * 0.5


def _tables(x, a, b):
    n = x.shape[0]
    return pl.pallas_call(
        _tables_body,
        out_shape=(
            jax.ShapeDtypeStruct((n, H), jnp.float32),
            jax.ShapeDtypeStruct((n, H), jnp.float32),
            jax.ShapeDtypeStruct((n, H), jnp.float32),
        ),
    )(x, a, b)


def _efc_body(ef_ref, c_ref, b_ref, o_ref):
    o_ref[...] = (
        jnp.dot(ef_ref[...], c_ref[...], preferred_element_type=jnp.float32)
        + b_ref[...]
    )


def _efc(ef, c, b1):
    e = ef.shape[0]
    be = 4000
    grid = e // be
    return pl.pallas_call(
        _efc_body,
        grid=(grid,),
        in_specs=[
            pl.BlockSpec((be, H), lambda i: (i, 0)),
            pl.BlockSpec((H, H), lambda i: (0, 0)),
            pl.BlockSpec((1, H), lambda i: (0, 0)),
        ],
        out_specs=pl.BlockSpec((be, H), lambda i: (i, 0)),
        out_shape=jax.ShapeDtypeStruct((e, H), jnp.float32),
    )(ef, c, b1.reshape(1, H))


def _sc_body(xa, xb, efc, sr_hbm, xh_hbm, out_hbm,
             sr0, sr1, sr2, eidx0, eidx1, eidx2, rsc0, rsc1, rsc2,
             acc0, acc1, acc2, aggr,
             semi0, semi1, semi2, sema0, sema1, sema2,
             semb0, semb1, semb2, semc0, semc1, semc2, ssem):
    c = lax.axis_index("c")
    s = lax.axis_index("s")
    wid = c * NS + s
    srs = (sr0, sr1, sr2)
    eidxs = (eidx0, eidx1, eidx2)
    rscs = (rsc0, rsc1, rsc2)
    accs = (acc0, acc1, acc2)
    semis = (semi0, semi1, semi2)
    semas = (sema0, sema1, sema2)
    sembs = (semb0, semb1, semb2)
    semcs = (semc0, semc1, semc2)

    row0 = s * RPT
    pltpu.async_copy(xh_hbm.at[pl.ds(row0, RPT)], aggr.at[pl.ds(row0, RPT)],
                     ssem).wait()

    @pl.when(s == 0)
    def _seed_tail():
        pltpu.async_copy(xh_hbm.at[pl.ds(TAIL0, TAILN)],
                         aggr.at[pl.ds(TAIL0, TAILN)], ssem).wait()

    plsc.subcore_barrier()

    nchunk_w = NCH_BASE + jnp.where(wid < NCH_REM, 1, 0)
    lanes = lax.iota(jnp.int32, L)

    def ebase(k):
        return (k * NW + wid) * ECH

    def issue_idx(k, i):
        pltpu.async_copy(sr_hbm.at[:, pl.ds(ebase(k), ECH)], srs[i], semis[i])

    def wait_idx(k, i):
        pltpu.make_async_copy(sr_hbm.at[:, pl.ds(ebase(k), ECH)], srs[i],
                              semis[i]).wait()

    def issue_xa(k, i):
        pltpu.async_copy(xa.at[srs[i].at[0]], accs[i], semas[i])

    def wait_xa(k, i):
        pltpu.make_async_copy(xa.at[srs[i].at[0]], accs[i], semas[i]).wait()

    def issue_adds(k, i):
        base = ebase(k)
        for j in range(ECH // L):
            eidxs[i][pl.ds(j * L, L)] = base + j * L + lanes
        pltpu.async_copy(xb.at[srs[i].at[1]], accs[i], sembs[i], add=True)
        pltpu.async_copy(efc.at[eidxs[i]], accs[i], sembs[i], add=True)

    def wait_adds(k, i):
        pltpu.make_async_copy(xb.at[srs[i].at[1]], accs[i], sembs[i]).wait()
        pltpu.make_async_copy(efc.at[eidxs[i]], accs[i], sembs[i]).wait()

    def wait_scatter(i):
        pltpu.make_async_copy(accs[i], aggr.at[rscs[i]], semcs[i]).wait()

    issue_idx(0, 0)
    issue_idx(1, 1)
    issue_idx(2, 2)
    wait_idx(0, 0)
    issue_xa(0, 0)
    wait_xa(0, 0)
    issue_adds(0, 0)
    wait_idx(1, 1)
    issue_xa(1, 1)

    def sub_body(k, i):
        i1 = (i + 1) % 3
        i2 = (i + 2) % 3

        @pl.when(k + 2 < nchunk_w)
        def _start_xa():
            wait_idx(k + 2, i2)

            @pl.when(k >= 1)
            def _prev_scatter_done():
                wait_scatter(i2)

            issue_xa(k + 2, i2)

        @pl.when(k + 1 < nchunk_w)
        def _start_adds():
            wait_xa(k + 1, i1)
            issue_adds(k + 1, i1)

        wait_adds(k, i)
        acc_i = accs[i]
        rsc_i = rscs[i]

        for j in range(ECH // L):
            rsc_i[pl.ds(j * L, L)] = srs[i][1, pl.ds(j * L, L)]

        def row(rr, carry2):
            for j in range(H // L):
                sl = pl.ds(j * L, L)
                acc_i[rr, sl] = jnp.maximum(acc_i[rr, sl], 0.0)
            return carry2

        lax.fori_loop(0, ECH, row, 0, unroll=2)
        pltpu.async_copy(acc_i, aggr.at[rsc_i], semcs[i], add=True)

        @pl.when(k + 3 < nchunk_w)
        def _refill_idx():
            issue_idx(k + 3, i)

    def triple(t, carry):
        k = t * 3

        @pl.when(k < nchunk_w)
        def _first():
            sub_body(k, 0)

        @pl.when(k + 1 < nchunk_w)
        def _mid():
            sub_body(k + 1, 1)

        @pl.when(k + 2 < nchunk_w)
        def _last():
            sub_body(k + 2, 2)

        return carry

    lax.fori_loop(0, NTRIP, triple, 0)

    wait_scatter(0)
    wait_scatter(1)
    wait_scatter(2)

    plsc.subcore_barrier()
    pltpu.async_copy(aggr.at[pl.ds(row0, RPT)],
                     out_hbm.at[c, pl.ds(row0, RPT)], ssem).wait()

    @pl.when(s == 0)
    def _write_tail():
        pltpu.async_copy(aggr.at[pl.ds(TAIL0, TAILN)],
                         out_hbm.at[c, pl.ds(TAIL0, TAILN)], ssem).wait()


def _sc_stage(xa, xb, efc, sr, xh):
    mesh = plsc.VectorSubcoreMesh(core_axis_name="c", subcore_axis_name="s")
    kfun = pl.kernel(
        _sc_body,
        out_type=jax.ShapeDtypeStruct((NC, N_NODES, H), jnp.float32),
        mesh=mesh,
        scratch_types=(
            [pltpu.VMEM((2, ECH), jnp.int32)] * 3
            + [pltpu.VMEM((ECH,), jnp.int32)] * 6
            + [pltpu.VMEM((ECH, H), jnp.float32)] * 3
            + [pltpu.VMEM_SHARED((N_NODES, H), jnp.float32)]
            + [pltpu.SemaphoreType.DMA] * 13
        ),
    )
    return kfun(xa, xb, efc, sr, xh)


def _combine_body(p_ref, o_ref):
    o_ref[...] = p_ref[0] + p_ref[1]


def _combine(partials):
    bn = 2000
    grid = N_NODES // bn
    return pl.pallas_call(
        _combine_body,
        grid=(grid,),
        in_specs=[pl.BlockSpec((NC, bn, H), lambda i: (0, i, 0))],
        out_specs=pl.BlockSpec((bn, H), lambda i: (i, 0)),
        out_shape=jax.ShapeDtypeStruct((N_NODES, H), jnp.float32),
    )(partials)


def kernel(x, senders, receivers, edge_feat, W1, b1):
    sr = jnp.stack([senders.astype(jnp.int32), receivers.astype(jnp.int32)])
    a = W1[:H]
    b = W1[H:2 * H]
    c = W1[2 * H:]
    xa, xb, xh = _tables(x, a, b)
    efc = _efc(edge_feat, c, b1)
    partials = _sc_stage(xa, xb, efc, sr, xh)
    return _combine(partials)


# speedup vs baseline: 1.3019x; 1.0402x over previous
import jax
import jax.numpy as jnp
from jax import lax
from jax.experimental import pallas as pl
from jax.experimental.pallas import tpu as pltpu
from jax.experimental.pallas import tpu_sc as plsc

N_NODES = 10000
N_EDGES = 320000
H = 128

NC = 2
NS = 16
L = 16
NW = NC * NS
ECH = 128
NCH_ALL = N_EDGES // ECH
NCH_BASE = NCH_ALL // NW
NCH_REM = NCH_ALL % NW
NTRIP = (NCH_BASE + 1 + 2) // 3
RPT = 624
TAIL0 = NS * RPT
TAILN = N_NODES - TAIL0


def _tables_body(x_ref, a_ref, b_ref, xa_ref, xb_ref, xh_ref):
    x = x_ref[...]
    xa_ref[...] = jnp.dot(x, a_ref[...], preferred_element_type=jnp.float32)
    xb_ref[...] = jnp.dot(x, b_ref[...], preferred_element_type=jnp.float32)
    xh_ref[...] = x * 0.5


def _tables(x, a, b):
    n = x.shape[0]
    return pl.pallas_call(
        _tables_body,
        out_shape=(
            jax.ShapeDtypeStruct((n, H), jnp.float32),
            jax.ShapeDtypeStruct((n, H), jnp.float32),
            jax.ShapeDtypeStruct((n, H), jnp.float32),
        ),
    )(x, a, b)


def _efc_body(ef_ref, c_ref, b_ref, o_ref):
    o_ref[...] = (
        jnp.dot(ef_ref[...], c_ref[...], preferred_element_type=jnp.float32)
        + b_ref[...]
    )


def _efc(ef, c, b1):
    e = ef.shape[0]
    be = 4000
    grid = e // be
    return pl.pallas_call(
        _efc_body,
        grid=(grid,),
        in_specs=[
            pl.BlockSpec((be, H), lambda i: (i, 0)),
            pl.BlockSpec((H, H), lambda i: (0, 0)),
            pl.BlockSpec((1, H), lambda i: (0, 0)),
        ],
        out_specs=pl.BlockSpec((be, H), lambda i: (i, 0)),
        out_shape=jax.ShapeDtypeStruct((e, H), jnp.float32),
    )(ef, c, b1.reshape(1, H))


def _sc_body(xa, xb, efc, s_hbm, r_hbm, xh_hbm, out_hbm,
             sidx0, sidx1, sidx2, ridx0, ridx1, ridx2,
             eidx0, eidx1, eidx2, rsc0, rsc1, rsc2,
             acc0, acc1, acc2, aggr,
             semi0, semi1, semi2, sema0, sema1, sema2,
             semb0, semb1, semb2, semc0, semc1, semc2, ssem):
    c = lax.axis_index("c")
    s = lax.axis_index("s")
    wid = c * NS + s
    sidxs = (sidx0, sidx1, sidx2)
    ridxs = (ridx0, ridx1, ridx2)
    eidxs = (eidx0, eidx1, eidx2)
    rscs = (rsc0, rsc1, rsc2)
    accs = (acc0, acc1, acc2)
    semis = (semi0, semi1, semi2)
    semas = (sema0, sema1, sema2)
    sembs = (semb0, semb1, semb2)
    semcs = (semc0, semc1, semc2)

    row0 = s * RPT
    pltpu.async_copy(xh_hbm.at[pl.ds(row0, RPT)], aggr.at[pl.ds(row0, RPT)],
                     ssem).wait()

    @pl.when(s == 0)
    def _seed_tail():
        pltpu.async_copy(xh_hbm.at[pl.ds(TAIL0, TAILN)],
                         aggr.at[pl.ds(TAIL0, TAILN)], ssem).wait()

    plsc.subcore_barrier()

    nchunk_w = NCH_BASE + jnp.where(wid < NCH_REM, 1, 0)
    lanes = lax.iota(jnp.int32, L)

    def ebase(k):
        return (k * NW + wid) * ECH

    def issue_idx(k, i):
        pltpu.async_copy(s_hbm.at[pl.ds(ebase(k), ECH)], sidxs[i], semis[i])
        pltpu.async_copy(r_hbm.at[pl.ds(ebase(k), ECH)], ridxs[i], semis[i])

    def wait_idx(k, i):
        pltpu.make_async_copy(s_hbm.at[pl.ds(ebase(k), ECH)], sidxs[i],
                              semis[i]).wait()
        pltpu.make_async_copy(r_hbm.at[pl.ds(ebase(k), ECH)], ridxs[i],
                              semis[i]).wait()

    def issue_xa(k, i):
        pltpu.async_copy(xa.at[sidxs[i]], accs[i], semas[i])

    def wait_xa(k, i):
        pltpu.make_async_copy(xa.at[sidxs[i]], accs[i], semas[i]).wait()

    def issue_adds(k, i):
        base = ebase(k)
        for j in range(ECH // L):
            eidxs[i][pl.ds(j * L, L)] = base + j * L + lanes
        pltpu.async_copy(xb.at[ridxs[i]], accs[i], sembs[i], add=True)
        pltpu.async_copy(efc.at[eidxs[i]], accs[i], sembs[i], add=True)

    def wait_adds(k, i):
        pltpu.make_async_copy(xb.at[ridxs[i]], accs[i], sembs[i]).wait()
        pltpu.make_async_copy(efc.at[eidxs[i]], accs[i], sembs[i]).wait()

    def wait_scatter(i):
        pltpu.make_async_copy(accs[i], aggr.at[rscs[i]], semcs[i]).wait()

    issue_idx(0, 0)
    issue_idx(1, 1)
    issue_idx(2, 2)
    wait_idx(0, 0)
    issue_xa(0, 0)
    wait_xa(0, 0)
    issue_adds(0, 0)
    wait_idx(1, 1)
    issue_xa(1, 1)

    def sub_body(k, i):
        i1 = (i + 1) % 3
        i2 = (i + 2) % 3

        @pl.when(k + 2 < nchunk_w)
        def _start_xa():
            wait_idx(k + 2, i2)

            @pl.when(k >= 1)
            def _prev_scatter_done():
                wait_scatter(i2)

            issue_xa(k + 2, i2)

        @pl.when(k + 1 < nchunk_w)
        def _start_adds():
            wait_xa(k + 1, i1)
            issue_adds(k + 1, i1)

        wait_adds(k, i)
        acc_i = accs[i]
        rsc_i = rscs[i]

        for j in range(ECH // L):
            rsc_i[pl.ds(j * L, L)] = ridxs[i][pl.ds(j * L, L)]

        def row(rr, carry2):
            for j in range(H // L):
                sl = pl.ds(j * L, L)
                acc_i[rr, sl] = jnp.maximum(acc_i[rr, sl], 0.0)
            return carry2

        lax.fori_loop(0, ECH, row, 0, unroll=2)
        pltpu.async_copy(acc_i, aggr.at[rsc_i], semcs[i], add=True)

        @pl.when(k + 3 < nchunk_w)
        def _refill_idx():
            issue_idx(k + 3, i)

    def triple(t, carry):
        k = t * 3

        @pl.when(k < nchunk_w)
        def _first():
            sub_body(k, 0)

        @pl.when(k + 1 < nchunk_w)
        def _mid():
            sub_body(k + 1, 1)

        @pl.when(k + 2 < nchunk_w)
        def _last():
            sub_body(k + 2, 2)

        return carry

    lax.fori_loop(0, NTRIP, triple, 0)

    wait_scatter(0)
    wait_scatter(1)
    wait_scatter(2)

    plsc.subcore_barrier()
    pltpu.async_copy(aggr.at[pl.ds(row0, RPT)],
                     out_hbm.at[c, pl.ds(row0, RPT)], ssem).wait()

    @pl.when(s == 0)
    def _write_tail():
        pltpu.async_copy(aggr.at[pl.ds(TAIL0, TAILN)],
                         out_hbm.at[c, pl.ds(TAIL0, TAILN)], ssem).wait()


def _sc_stage(xa, xb, efc, senders, receivers, xh):
    mesh = plsc.VectorSubcoreMesh(core_axis_name="c", subcore_axis_name="s")
    kfun = pl.kernel(
        _sc_body,
        out_type=jax.ShapeDtypeStruct((NC, N_NODES, H), jnp.float32),
        mesh=mesh,
        scratch_types=(
            [pltpu.VMEM((ECH,), jnp.int32)] * 12
            + [pltpu.VMEM((ECH, H), jnp.float32)] * 3
            + [pltpu.VMEM_SHARED((N_NODES, H), jnp.float32)]
            + [pltpu.SemaphoreType.DMA] * 13
        ),
    )
    return kfun(xa, xb, efc, senders, receivers, xh)


def _combine_body(p_ref, o_ref):
    o_ref[...] = p_ref[0] + p_ref[1]


def _combine(partials):
    bn = 2000
    grid = N_NODES // bn
    return pl.pallas_call(
        _combine_body,
        grid=(grid,),
        in_specs=[pl.BlockSpec((NC, bn, H), lambda i: (0, i, 0))],
        out_specs=pl.BlockSpec((bn, H), lambda i: (i, 0)),
        out_shape=jax.ShapeDtypeStruct((N_NODES, H), jnp.float32),
    )(partials)


def kernel(x, senders, receivers, edge_feat, W1, b1):
    senders = senders.astype(jnp.int32)
    receivers = receivers.astype(jnp.int32)
    a = W1[:H]
    b = W1[H:2 * H]
    c = W1[2 * H:]
    xa, xb, xh = _tables(x, a, b)
    efc = _efc(edge_feat, c, b1)
    partials = _sc_stage(xa, xb, efc, senders, receivers, xh)
    return _combine(partials)
